# baseline (speedup 1.0000x reference)
import functools

import jax
import jax.numpy as jnp
from jax import lax
from jax.experimental import pallas as pl
from jax.experimental.pallas import tpu as pltpu
from jax.experimental.pallas import tpu_sc as plsc

B = 2
S = 4096
D = 2048
HID = 4 * D
K = 491
KP = 512
NR = B * KP
NC = 2
NS = 16
NW = NC * NS
RPW = NR // NW
CPW = (B * S) // NW
NBLK = 8
SB = S // NBLK
HBLK = 16
HB = HID // HBLK
NEG_INF = float("-inf")
_I = False


def _router_body(x_ref, wra_ref, lg_ref, alog_ref):
    i = pl.program_id(0)
    xb = x_ref[...].reshape(B * SB, D).astype(jnp.bfloat16)
    y = jnp.dot(xb, wra_ref[...], preferred_element_type=jnp.float32)
    lg_ref[:, pl.ds(i * SB, SB)] = y[:, 0].reshape(B, SB)
    alog_ref[:, pl.ds(i * SB, SB)] = y[:, 1].reshape(B, SB)


def _router(x, wra):
    return pl.pallas_call(
        _router_body,
        grid=(NBLK,),
        in_specs=[
            pl.BlockSpec((B, SB, D), lambda i: (0, i, 0)),
            pl.BlockSpec((D, 128), lambda i: (0, 0)),
        ],
        out_specs=[
            pl.BlockSpec((B, S), lambda i: (0, 0)),
            pl.BlockSpec((B, S), lambda i: (0, 0)),
        ],
        out_shape=[
            jax.ShapeDtypeStruct((B, S), jnp.float32),
            jax.ShapeDtypeStruct((B, S), jnp.float32),
        ],
        compiler_params=pltpu.CompilerParams(
            dimension_semantics=("arbitrary",)),
        interpret=_I,
    )(x, wra)


def _excl_prefix(m):
    tri128 = (lax.broadcasted_iota(jnp.int32, (128, 128), 0)
              < lax.broadcasted_iota(jnp.int32, (128, 128), 1)).astype(jnp.float32)
    in_row = jax.lax.dot(m, tri128, precision=lax.Precision.HIGHEST)
    row_tot = jnp.sum(m, axis=1, keepdims=True)
    tri32 = (lax.broadcasted_iota(jnp.int32, (32, 32), 0)
             < lax.broadcasted_iota(jnp.int32, (32, 32), 1)).astype(jnp.float32)
    row_off = jax.lax.dot(row_tot.reshape(1, 32), tri32,
                          precision=lax.Precision.HIGHEST)
    return in_row + row_off.reshape(32, 1)


def _topk_body(lg_ref, alog_ref, sel_ref, w_ref, aux_ref):
    gif = (lax.broadcasted_iota(jnp.int32, (32, 128), 0) * 128
           + lax.broadcasted_iota(jnp.int32, (32, 128), 1)).astype(jnp.float32)
    piota = lax.broadcasted_iota(jnp.int32, (KP, 1), 0).astype(jnp.float32)
    piota_row = lax.broadcasted_iota(jnp.int32, (1, KP), 1).astype(jnp.float32)
    jcol = lax.broadcasted_iota(jnp.int32, (KP, 1), 0)
    pmask = (piota < K)
    pmask_row = (piota_row < K)

    masks = []
    for b in range(B):
        v = lg_ref[b, :].reshape(32, 128)
        s = lax.bitcast_convert_type(v, jnp.int32)
        key = s ^ ((s >> 31) & jnp.int32(0x7FFFFFFF))

        def bs_step(_, carry):
            lo, hi = carry
            mid = (lo >> 1) + (hi >> 1) + ((lo | hi) & 1)
            cnt = jnp.sum((key >= mid).astype(jnp.int32))
            big = cnt >= K
            return (jnp.where(big, mid, lo), jnp.where(big, hi, mid - 1))

        lo0 = jnp.int32(-2147483647 - 1)
        hi0 = jnp.int32(2147483647)
        thr, _ = lax.fori_loop(0, 32, bs_step, (lo0, hi0))

        gt = (key > thr).astype(jnp.float32)
        ties = (key == thr).astype(jnp.float32)
        m_rem = (jnp.float32(K) - jnp.sum(gt)).astype(jnp.float32)
        tie_pref = _excl_prefix(ties)
        m = gt + ties * (tie_pref < m_rem).astype(jnp.float32)
        masks.append(m)

        pos = _excl_prefix(m)
        u = jnp.min(jnp.where(m == 0, gif, jnp.float32(S)))
        posr = pos.reshape(1, S)
        mr = m.reshape(1, S)
        gir = gif.reshape(1, S)
        vr = v.reshape(1, S)
        sel_asc = jnp.zeros((KP, 1), jnp.float32)
        kv = jnp.zeros((KP, 1), jnp.float32)
        CS = 512
        for ci in range(S // CS):
            lo, hi = ci * CS, (ci + 1) * CS
            Ac = ((posr[:, lo:hi] == piota).astype(jnp.float32)
                  * mr[:, lo:hi])
            sel_asc = sel_asc + jnp.sum(Ac * gir[:, lo:hi], axis=1,
                                        keepdims=True)
            kv = kv + jnp.sum(Ac * vr[:, lo:hi], axis=1, keepdims=True)
        sel_p = jnp.where(pmask, sel_asc, u)
        vsel = jnp.where(pmask, kv, jnp.float32(NEG_INF))
        vrow = jnp.transpose(vsel)
        rank = jnp.zeros((KP, 1), jnp.float32)
        for qc in range(KP // 128):
            vq = vrow[:, qc * 128:(qc + 1) * 128]
            qid = (lax.broadcasted_iota(jnp.int32, (1, 128), 1)
                   + qc * 128)
            cmp = ((vq > vsel) | ((vq == vsel) & (qid < jcol)))
            rank = rank + jnp.sum(cmp.astype(jnp.float32), axis=1,
                                  keepdims=True)
        wrow = jnp.zeros((1, KP), jnp.float32)
        for jc in range(KP // 128):
            rj = rank[jc * 128:(jc + 1) * 128, :]
            vj = vsel[jc * 128:(jc + 1) * 128, :]
            oh = (rj == piota_row).astype(jnp.float32)
            wrow = wrow + jnp.sum(oh * vj, axis=0, keepdims=True)
        wv = jnp.where(pmask_row, wrow, 0.0)
        sel_row = jnp.transpose(sel_p).astype(jnp.int32) + b * S
        sel_ref[pl.ds(b, 1), :] = sel_row
        w_ref[pl.ds(b, 1), :] = wv

    al = alog_ref[...]
    p = jnp.clip(jax.nn.sigmoid(al), 1e-7, 1.0 - 1e-7)
    logp = jnp.log(p)
    log1mp = jnp.log(1.0 - p)
    base = jnp.sum(log1mp)
    mu = jnp.minimum(masks[0] + masks[1], 1.0)
    c0 = (logp[0, :] - log1mp[0, :]).reshape(32, 128)
    corr = jnp.sum(mu * c0)
    aux_ref[...] = (-(base + corr) / jnp.float32(B * S)).reshape(1, 1)


def _topk(lg, alog):
    return pl.pallas_call(
        _topk_body,
        out_shape=[
            jax.ShapeDtypeStruct((B, KP), jnp.int32),
            jax.ShapeDtypeStruct((B, KP), jnp.float32),
            jax.ShapeDtypeStruct((1, 1), jnp.float32),
        ],
        interpret=_I,
    )(lg, alog)


@functools.lru_cache(maxsize=None)
def _sc_kernels():
    mesh = plsc.VectorSubcoreMesh(core_axis_name="c", subcore_axis_name="s",
                                  num_cores=NC, num_subcores=NS)

    @functools.partial(
        pl.kernel,
        out_type=jax.ShapeDtypeStruct((NR, D), jnp.float32),
        mesh=mesh,
        scratch_types=[
            pltpu.VMEM((RPW,), jnp.int32),
            pltpu.VMEM((RPW, D), jnp.float32),
            pltpu.SemaphoreType.DMA,
        ],
    )
    def gather_k(x2d, idx2d, fx, idx_v, rows_v, sem):
        w = lax.axis_index("c") * NS + lax.axis_index("s")
        pltpu.sync_copy(idx2d.at[w], idx_v)
        pltpu.async_copy(x2d.at[idx_v], rows_v, sem).wait()
        pltpu.sync_copy(rows_v, fx.at[pl.ds(w * RPW, RPW)])

    @functools.partial(
        pl.kernel,
        out_type=jax.ShapeDtypeStruct((B * S, D), jnp.float32),
        mesh=mesh,
        scratch_types=[
            pltpu.VMEM((RPW,), jnp.int32),
            pltpu.VMEM((RPW, D), jnp.float32),
            pltpu.SemaphoreType.DMA,
        ],
    )
    def scatter_k(x2d, rows, idx2d, out, idx_v, buf, sem):
        c = lax.axis_index("c")
        w = c * NS + lax.axis_index("s")
        base = w * CPW
        pltpu.sync_copy(x2d.at[pl.ds(base, CPW)], out.at[pl.ds(base, CPW)])
        plsc.subcore_barrier()
        pltpu.sync_copy(idx2d.at[w], idx_v)
        pltpu.sync_copy(rows.at[pl.ds(w * RPW, RPW)], buf)
        pltpu.async_copy(buf, out.at[idx_v], sem).wait()

    return gather_k, scatter_k


def _gather(x2d, idx2d):
    return _sc_kernels()[0](x2d, idx2d)


def _scatter(x2d, rows, idx2d):
    return _sc_kernels()[1](x2d, rows, idx2d)


def _mlp_body(fx_ref, w_ref, w1_ref, w3_ref, w2_ref, out_ref, fxb_ref):
    h = pl.program_id(0)

    @pl.when(h == 0)
    def _init():
        fxb_ref[...] = fx_ref[...].astype(jnp.bfloat16)
        out_ref[...] = jnp.zeros_like(out_ref)

    fxb = fxb_ref[...]
    h1 = jnp.dot(fxb, w1_ref[...].astype(jnp.bfloat16),
                 preferred_element_type=jnp.float32)
    h3 = jnp.dot(fxb, w3_ref[...].astype(jnp.bfloat16),
                 preferred_element_type=jnp.float32)
    g = (h1 * jax.nn.sigmoid(h1) * h3).astype(jnp.bfloat16)
    out_ref[...] += jnp.dot(g, w2_ref[...].astype(jnp.bfloat16),
                            preferred_element_type=jnp.float32)

    @pl.when(h == HBLK - 1)
    def _fin():
        out_ref[...] = fx_ref[...] + w_ref[...] * out_ref[...]


def _mlp(fx, wrow, w1, w3, w2):
    return pl.pallas_call(
        _mlp_body,
        grid=(HBLK,),
        in_specs=[
            pl.BlockSpec((NR, D), lambda h: (0, 0)),
            pl.BlockSpec((NR, 1), lambda h: (0, 0)),
            pl.BlockSpec((D, HB), lambda h: (0, h)),
            pl.BlockSpec((D, HB), lambda h: (0, h)),
            pl.BlockSpec((HB, D), lambda h: (h, 0)),
        ],
        out_specs=pl.BlockSpec((NR, D), lambda h: (0, 0)),
        out_shape=jax.ShapeDtypeStruct((NR, D), jnp.float32),
        scratch_shapes=[pltpu.VMEM((NR, D), jnp.bfloat16)],
        compiler_params=pltpu.CompilerParams(
            dimension_semantics=("arbitrary",)),
        interpret=_I,
    )(fx, wrow, w1, w3, w2)


def kernel(x, Wr, Wa, W1, W2, W3):
    wra = jnp.concatenate(
        [Wr, Wa, jnp.zeros((D, 126), jnp.float32)], axis=1).astype(jnp.bfloat16)
    lg, alog = _router(x, wra)
    sel8, w8, aux = _topk(lg, alog)
    idx2d = sel8.reshape(NW, RPW)
    wrow = w8.reshape(NR, 1)
    x2d = x.reshape(B * S, D)
    fx = _gather(x2d, idx2d)
    rows = _mlp(fx, wrow, W1, W3, W2)
    out2d = _scatter(x2d, rows, idx2d)
    return out2d.reshape(B, S, D), aux.reshape(())


# speedup vs baseline: 8.7738x; 8.7738x over previous
import functools

import jax
import jax.numpy as jnp
from jax import lax
from jax.experimental import pallas as pl
from jax.experimental.pallas import tpu as pltpu
from jax.experimental.pallas import tpu_sc as plsc

B = 2
S = 4096
D = 2048
HID = 4 * D
K = 491
KP = 512
NR = B * KP
NC = 2
NS = 16
NW = NC * NS
RPW = NR // NW
CPW = (B * S) // NW
NBLK = 8
SB = S // NBLK
HBLK = 16
HB = HID // HBLK
NEG_INF = float("-inf")
_I = False


def _router_body(x_ref, wra_ref, lg_ref, alog_ref):
    i = pl.program_id(0)
    xb = x_ref[...].reshape(B * SB, D).astype(jnp.bfloat16)
    y = jnp.dot(xb, wra_ref[...], preferred_element_type=jnp.float32)
    lg_ref[:, pl.ds(i * SB, SB)] = y[:, 0].reshape(B, SB)
    alog_ref[:, pl.ds(i * SB, SB)] = y[:, 1].reshape(B, SB)


def _router(x, wra):
    return pl.pallas_call(
        _router_body,
        grid=(NBLK,),
        in_specs=[
            pl.BlockSpec((B, SB, D), lambda i: (0, i, 0)),
            pl.BlockSpec((D, 128), lambda i: (0, 0)),
        ],
        out_specs=[
            pl.BlockSpec((B, S), lambda i: (0, 0)),
            pl.BlockSpec((B, S), lambda i: (0, 0)),
        ],
        out_shape=[
            jax.ShapeDtypeStruct((B, S), jnp.float32),
            jax.ShapeDtypeStruct((B, S), jnp.float32),
        ],
        compiler_params=pltpu.CompilerParams(
            dimension_semantics=("arbitrary",)),
        interpret=_I,
    )(x, wra)


def _excl_prefix(m):
    tri128 = (lax.broadcasted_iota(jnp.int32, (128, 128), 0)
              < lax.broadcasted_iota(jnp.int32, (128, 128), 1)).astype(jnp.float32)
    in_row = jax.lax.dot(m, tri128, precision=lax.Precision.HIGHEST)
    row_tot = jnp.sum(m, axis=1, keepdims=True)
    tri32 = (lax.broadcasted_iota(jnp.int32, (32, 32), 0)
             < lax.broadcasted_iota(jnp.int32, (32, 32), 1)).astype(jnp.float32)
    row_off = jax.lax.dot(row_tot.reshape(1, 32), tri32,
                          precision=lax.Precision.HIGHEST)
    return in_row + row_off.reshape(32, 1)


def _topk_body(lg_ref, alog_ref, sel_ref, w_ref, aux_ref):
    gif = (lax.broadcasted_iota(jnp.int32, (32, 128), 0) * 128
           + lax.broadcasted_iota(jnp.int32, (32, 128), 1)).astype(jnp.float32)
    piota = lax.broadcasted_iota(jnp.int32, (KP, 1), 0).astype(jnp.float32)
    piota_row = lax.broadcasted_iota(jnp.int32, (1, KP), 1).astype(jnp.float32)
    jcol = lax.broadcasted_iota(jnp.int32, (KP, 1), 0)
    pmask = (piota < K)
    pmask_row = (piota_row < K)

    masks = []
    for b in range(B):
        v = lg_ref[b, :].reshape(32, 128)
        s = lax.bitcast_convert_type(v, jnp.int32)
        key = s ^ ((s >> 31) & jnp.int32(0x7FFFFFFF))

        def bs_step(_, carry):
            lo, hi = carry
            mid = (lo >> 1) + (hi >> 1) + ((lo | hi) & 1)
            cnt = jnp.sum((key >= mid).astype(jnp.int32))
            big = cnt >= K
            return (jnp.where(big, mid, lo), jnp.where(big, hi, mid - 1))

        lo0 = jnp.int32(-2147483647 - 1)
        hi0 = jnp.int32(2147483647)
        thr, _ = lax.fori_loop(0, 32, bs_step, (lo0, hi0))

        gt = (key > thr).astype(jnp.float32)
        ties = (key == thr).astype(jnp.float32)
        m_rem = (jnp.float32(K) - jnp.sum(gt)).astype(jnp.float32)
        tie_pref = _excl_prefix(ties)
        m = gt + ties * (tie_pref < m_rem).astype(jnp.float32)
        masks.append(m)

        pos = _excl_prefix(m)
        u = jnp.min(jnp.where(m == 0, gif, jnp.float32(S)))
        posr = pos.reshape(1, S)
        mr = m.reshape(1, S)
        gir = gif.reshape(1, S)
        vr = v.reshape(1, S)
        sel_asc = jnp.zeros((KP, 1), jnp.float32)
        kv = jnp.zeros((KP, 1), jnp.float32)
        CS = 512
        for ci in range(S // CS):
            lo, hi = ci * CS, (ci + 1) * CS
            Ac = ((posr[:, lo:hi] == piota).astype(jnp.float32)
                  * mr[:, lo:hi])
            sel_asc = sel_asc + jnp.sum(Ac * gir[:, lo:hi], axis=1,
                                        keepdims=True)
            kv = kv + jnp.sum(Ac * vr[:, lo:hi], axis=1, keepdims=True)
        sel_p = jnp.where(pmask, sel_asc, u)
        vsel = jnp.where(pmask, kv, jnp.float32(NEG_INF))
        vrow = jnp.transpose(vsel)
        rank = jnp.zeros((KP, 1), jnp.float32)
        for qc in range(KP // 128):
            vq = vrow[:, qc * 128:(qc + 1) * 128]
            qid = (lax.broadcasted_iota(jnp.int32, (1, 128), 1)
                   + qc * 128)
            cmp = ((vq > vsel) | ((vq == vsel) & (qid < jcol)))
            rank = rank + jnp.sum(cmp.astype(jnp.float32), axis=1,
                                  keepdims=True)
        wrow = jnp.zeros((1, KP), jnp.float32)
        for jc in range(KP // 128):
            rj = rank[jc * 128:(jc + 1) * 128, :]
            vj = vsel[jc * 128:(jc + 1) * 128, :]
            oh = (rj == piota_row).astype(jnp.float32)
            wrow = wrow + jnp.sum(oh * vj, axis=0, keepdims=True)
        wv = jnp.where(pmask_row, wrow, 0.0)
        sel_row = jnp.transpose(sel_p).astype(jnp.int32) + b * S
        sel_ref[pl.ds(b, 1), :] = sel_row
        w_ref[pl.ds(b, 1), :] = wv

    al = alog_ref[...]
    p = jnp.clip(jax.nn.sigmoid(al), 1e-7, 1.0 - 1e-7)
    logp = jnp.log(p)
    log1mp = jnp.log(1.0 - p)
    base = jnp.sum(log1mp)
    mu = jnp.minimum(masks[0] + masks[1], 1.0)
    c0 = (logp[0, :] - log1mp[0, :]).reshape(32, 128)
    corr = jnp.sum(mu * c0)
    aux_ref[...] = (-(base + corr) / jnp.float32(B * S)).reshape(1, 1)


def _topk(lg, alog):
    return pl.pallas_call(
        _topk_body,
        out_shape=[
            jax.ShapeDtypeStruct((B, KP), jnp.int32),
            jax.ShapeDtypeStruct((B, KP), jnp.float32),
            jax.ShapeDtypeStruct((1, 1), jnp.float32),
        ],
        interpret=_I,
    )(lg, alog)


@functools.lru_cache(maxsize=None)
def _sc_kernels():
    mesh = plsc.VectorSubcoreMesh(core_axis_name="c", subcore_axis_name="s",
                                  num_cores=NC, num_subcores=NS)

    @functools.partial(
        pl.kernel,
        out_type=jax.ShapeDtypeStruct((NR, D), jnp.float32),
        mesh=mesh,
        scratch_types=[
            pltpu.VMEM((RPW,), jnp.int32),
            pltpu.VMEM((RPW, D), jnp.float32),
            pltpu.SemaphoreType.DMA,
        ],
    )
    def gather_k(x2d, idx2d, fx, idx_v, rows_v, sem):
        w = lax.axis_index("c") * NS + lax.axis_index("s")
        pltpu.sync_copy(idx2d.at[w], idx_v)
        pltpu.async_copy(x2d.at[idx_v], rows_v, sem).wait()
        pltpu.sync_copy(rows_v, fx.at[pl.ds(w * RPW, RPW)])

    @functools.partial(
        pl.kernel,
        out_type=jax.ShapeDtypeStruct((B * S, D), jnp.float32),
        mesh=mesh,
        scratch_types=[
            pltpu.VMEM((RPW,), jnp.int32),
            pltpu.VMEM((RPW, D), jnp.float32),
            pltpu.SemaphoreType.DMA,
        ],
    )
    def scatter_k(x2d, rows, idx2d, out, idx_v, buf, sem):
        c = lax.axis_index("c")
        w = c * NS + lax.axis_index("s")
        base = w * CPW
        for t in range(CPW // RPW):
            pltpu.sync_copy(x2d.at[pl.ds(base + t * RPW, RPW)], buf)
            pltpu.sync_copy(buf, out.at[pl.ds(base + t * RPW, RPW)])
        plsc.subcore_barrier()
        pltpu.sync_copy(idx2d.at[w], idx_v)
        pltpu.sync_copy(rows.at[pl.ds(w * RPW, RPW)], buf)
        pltpu.async_copy(buf, out.at[idx_v], sem).wait()

    return gather_k, scatter_k


def _gather(x2d, idx2d):
    return _sc_kernels()[0](x2d, idx2d)


def _scatter(x2d, rows, idx2d):
    return _sc_kernels()[1](x2d, rows, idx2d)


def _mlp_body(fx_ref, w_ref, w1_ref, w3_ref, w2_ref, out_ref, fxb_ref):
    h = pl.program_id(0)

    @pl.when(h == 0)
    def _init():
        fxb_ref[...] = fx_ref[...].astype(jnp.bfloat16)
        out_ref[...] = jnp.zeros_like(out_ref)

    fxb = fxb_ref[...]
    h1 = jnp.dot(fxb, w1_ref[...].astype(jnp.bfloat16),
                 preferred_element_type=jnp.float32)
    h3 = jnp.dot(fxb, w3_ref[...].astype(jnp.bfloat16),
                 preferred_element_type=jnp.float32)
    g = (h1 * jax.nn.sigmoid(h1) * h3).astype(jnp.bfloat16)
    out_ref[...] += jnp.dot(g, w2_ref[...].astype(jnp.bfloat16),
                            preferred_element_type=jnp.float32)

    @pl.when(h == HBLK - 1)
    def _fin():
        out_ref[...] = fx_ref[...] + w_ref[...] * out_ref[...]


def _mlp(fx, wrow, w1, w3, w2):
    return pl.pallas_call(
        _mlp_body,
        grid=(HBLK,),
        in_specs=[
            pl.BlockSpec((NR, D), lambda h: (0, 0)),
            pl.BlockSpec((NR, 1), lambda h: (0, 0)),
            pl.BlockSpec((D, HB), lambda h: (0, h)),
            pl.BlockSpec((D, HB), lambda h: (0, h)),
            pl.BlockSpec((HB, D), lambda h: (h, 0)),
        ],
        out_specs=pl.BlockSpec((NR, D), lambda h: (0, 0)),
        out_shape=jax.ShapeDtypeStruct((NR, D), jnp.float32),
        scratch_shapes=[pltpu.VMEM((NR, D), jnp.bfloat16)],
        compiler_params=pltpu.CompilerParams(
            dimension_semantics=("arbitrary",)),
        interpret=_I,
    )(fx, wrow, w1, w3, w2)


def kernel(x, Wr, Wa, W1, W2, W3):
    wra = jnp.concatenate(
        [Wr, Wa, jnp.zeros((D, 126), jnp.float32)], axis=1).astype(jnp.bfloat16)
    lg, alog = _router(x, wra)
    sel8, w8, aux = _topk(lg, alog)
    idx2d = sel8.reshape(NW, RPW)
    wrow = w8.reshape(NR, 1)
    x2d = x.reshape(B * S, D)
    fx = _gather(x2d, idx2d)
    rows = _mlp(fx, wrow, W1, W3, W2)
    out2d = _scatter(x2d, rows, idx2d)
    return out2d.reshape(B, S, D), aux.reshape(())


# speedup vs baseline: 8.7826x; 1.0010x over previous
import functools

import jax
import jax.numpy as jnp
from jax import lax
from jax.experimental import pallas as pl
from jax.experimental.pallas import tpu as pltpu
from jax.experimental.pallas import tpu_sc as plsc

B = 2
S = 4096
D = 2048
HID = 4 * D
K = 491
KP = 512
NR = B * KP
NC = 2
NS = 16
NW = NC * NS
RPW = NR // NW
CPW = (B * S) // NW
NBLK = 8
SB = S // NBLK
HBLK = 16
HB = HID // HBLK
NEG_INF = float("-inf")
_I = False


def _router_topk_body(x_ref, wra_ref, sel_ref, w_ref, aux_ref,
                      lg_s, alog_s):
    i = pl.program_id(0)

    @pl.when(i < NBLK)
    def _step():
        xb = x_ref[...].reshape(B * SB, D).astype(jnp.bfloat16)
        y = jnp.dot(xb, wra_ref[...], preferred_element_type=jnp.float32)
        lg_s[:, pl.ds(i * SB, SB)] = y[:, 0].reshape(B, SB)
        alog_s[:, pl.ds(i * SB, SB)] = y[:, 1].reshape(B, SB)

    @pl.when(i == NBLK)
    def _fin():
        _topk_compute(lg_s, alog_s, sel_ref, w_ref, aux_ref)


def _router_topk(x, wra):
    return pl.pallas_call(
        _router_topk_body,
        grid=(NBLK + 1,),
        in_specs=[
            pl.BlockSpec((B, SB, D), lambda i: (0, jnp.minimum(i, NBLK - 1), 0)),
            pl.BlockSpec((D, 128), lambda i: (0, 0)),
        ],
        out_specs=[
            pl.BlockSpec((B, KP), lambda i: (0, 0)),
            pl.BlockSpec((B, KP), lambda i: (0, 0)),
            pl.BlockSpec((1, 1), lambda i: (0, 0)),
        ],
        out_shape=[
            jax.ShapeDtypeStruct((B, KP), jnp.int32),
            jax.ShapeDtypeStruct((B, KP), jnp.float32),
            jax.ShapeDtypeStruct((1, 1), jnp.float32),
        ],
        scratch_shapes=[
            pltpu.VMEM((B, S), jnp.float32),
            pltpu.VMEM((B, S), jnp.float32),
        ],
        compiler_params=pltpu.CompilerParams(
            dimension_semantics=("arbitrary",)),
        interpret=_I,
    )(x, wra)


def _excl_prefix(m):
    tri128 = (lax.broadcasted_iota(jnp.int32, (128, 128), 0)
              < lax.broadcasted_iota(jnp.int32, (128, 128), 1)).astype(jnp.float32)
    in_row = jax.lax.dot(m, tri128, precision=lax.Precision.HIGHEST)
    row_tot = jnp.sum(m, axis=1, keepdims=True)
    tri32 = (lax.broadcasted_iota(jnp.int32, (32, 32), 0)
             < lax.broadcasted_iota(jnp.int32, (32, 32), 1)).astype(jnp.float32)
    row_off = jax.lax.dot(row_tot.reshape(1, 32), tri32,
                          precision=lax.Precision.HIGHEST)
    return in_row + row_off.reshape(32, 1)


def _topk_compute(lg_ref, alog_ref, sel_ref, w_ref, aux_ref):
    gif = (lax.broadcasted_iota(jnp.int32, (32, 128), 0) * 128
           + lax.broadcasted_iota(jnp.int32, (32, 128), 1)).astype(jnp.float32)
    piota = lax.broadcasted_iota(jnp.int32, (KP, 1), 0).astype(jnp.float32)
    piota_row = lax.broadcasted_iota(jnp.int32, (1, KP), 1).astype(jnp.float32)
    jcol = lax.broadcasted_iota(jnp.int32, (KP, 1), 0)
    pmask = (piota < K)
    pmask_row = (piota_row < K)

    masks = []
    for b in range(B):
        v = lg_ref[b, :].reshape(32, 128)
        s = lax.bitcast_convert_type(v, jnp.int32)
        key = s ^ ((s >> 31) & jnp.int32(0x7FFFFFFF))

        def bs_step(_, carry):
            lo, hi = carry
            mid = (lo >> 1) + (hi >> 1) + ((lo | hi) & 1)
            cnt = jnp.sum((key >= mid).astype(jnp.int32))
            big = cnt >= K
            return (jnp.where(big, mid, lo), jnp.where(big, hi, mid - 1))

        lo0 = jnp.int32(-2147483647 - 1)
        hi0 = jnp.int32(2147483647)
        thr, _ = lax.fori_loop(0, 32, bs_step, (lo0, hi0))

        gt = (key > thr).astype(jnp.float32)
        ties = (key == thr).astype(jnp.float32)
        m_rem = (jnp.float32(K) - jnp.sum(gt)).astype(jnp.float32)
        tie_pref = _excl_prefix(ties)
        m = gt + ties * (tie_pref < m_rem).astype(jnp.float32)
        masks.append(m)

        pos = _excl_prefix(m)
        u = jnp.min(jnp.where(m == 0, gif, jnp.float32(S)))
        posr = pos.reshape(1, S)
        mr = m.reshape(1, S)
        gir = gif.reshape(1, S)
        vr = v.reshape(1, S)
        sel_asc = jnp.zeros((KP, 1), jnp.float32)
        kv = jnp.zeros((KP, 1), jnp.float32)
        CS = 512
        for ci in range(S // CS):
            lo, hi = ci * CS, (ci + 1) * CS
            Ac = ((posr[:, lo:hi] == piota).astype(jnp.float32)
                  * mr[:, lo:hi])
            sel_asc = sel_asc + jnp.sum(Ac * gir[:, lo:hi], axis=1,
                                        keepdims=True)
            kv = kv + jnp.sum(Ac * vr[:, lo:hi], axis=1, keepdims=True)
        sel_p = jnp.where(pmask, sel_asc, u)
        vsel = jnp.where(pmask, kv, jnp.float32(NEG_INF))
        vrow = jnp.transpose(vsel)
        rank = jnp.zeros((KP, 1), jnp.float32)
        for qc in range(KP // 128):
            vq = vrow[:, qc * 128:(qc + 1) * 128]
            qid = (lax.broadcasted_iota(jnp.int32, (1, 128), 1)
                   + qc * 128)
            cmp = ((vq > vsel) | ((vq == vsel) & (qid < jcol)))
            rank = rank + jnp.sum(cmp.astype(jnp.float32), axis=1,
                                  keepdims=True)
        wrow = jnp.zeros((1, KP), jnp.float32)
        for jc in range(KP // 128):
            rj = rank[jc * 128:(jc + 1) * 128, :]
            vj = vsel[jc * 128:(jc + 1) * 128, :]
            oh = (rj == piota_row).astype(jnp.float32)
            wrow = wrow + jnp.sum(oh * vj, axis=0, keepdims=True)
        wv = jnp.where(pmask_row, wrow, 0.0)
        sel_row = jnp.transpose(sel_p).astype(jnp.int32) + b * S
        sel_ref[pl.ds(b, 1), :] = sel_row
        w_ref[pl.ds(b, 1), :] = wv

    al = alog_ref[...]
    p = jnp.clip(jax.nn.sigmoid(al), 1e-7, 1.0 - 1e-7)
    logp = jnp.log(p)
    log1mp = jnp.log(1.0 - p)
    base = jnp.sum(log1mp)
    mu = jnp.minimum(masks[0] + masks[1], 1.0)
    c0 = (logp[0, :] - log1mp[0, :]).reshape(32, 128)
    corr = jnp.sum(mu * c0)
    aux_ref[...] = (-(base + corr) / jnp.float32(B * S)).reshape(1, 1)


@functools.lru_cache(maxsize=None)
def _sc_kernels():
    mesh = plsc.VectorSubcoreMesh(core_axis_name="c", subcore_axis_name="s",
                                  num_cores=NC, num_subcores=NS)

    @functools.partial(
        pl.kernel,
        out_type=jax.ShapeDtypeStruct((NR, D), jnp.float32),
        mesh=mesh,
        scratch_types=[
            pltpu.VMEM((RPW,), jnp.int32),
            pltpu.VMEM((RPW, D), jnp.float32),
            pltpu.SemaphoreType.DMA,
        ],
    )
    def gather_k(x2d, idx2d, fx, idx_v, rows_v, sem):
        w = lax.axis_index("c") * NS + lax.axis_index("s")
        pltpu.sync_copy(idx2d.at[w], idx_v)
        pltpu.async_copy(x2d.at[idx_v], rows_v, sem).wait()
        pltpu.sync_copy(rows_v, fx.at[pl.ds(w * RPW, RPW)])

    @functools.partial(
        pl.kernel,
        out_type=jax.ShapeDtypeStruct((B * S, D), jnp.float32),
        mesh=mesh,
        scratch_types=[
            pltpu.VMEM((RPW // 2,), jnp.int32),
            pltpu.VMEM((RPW // 2,), jnp.int32),
            pltpu.VMEM((RPW // 2, D), jnp.float32),
            pltpu.VMEM((RPW // 2, D), jnp.float32),
            pltpu.SemaphoreType.DMA,
            pltpu.SemaphoreType.DMA,
        ],
    )
    def scatter_k(x2d, rows, idx2d, out, idx_a, idx_b, buf_a, buf_b,
                  sem_a, sem_b):
        c = lax.axis_index("c")
        w = c * NS + lax.axis_index("s")
        base = w * CPW
        half = RPW // 2
        pltpu.sync_copy(idx2d.at[w, pl.ds(0, half)], idx_a)
        pltpu.sync_copy(idx2d.at[w, pl.ds(half, half)], idx_b)
        bufs = (buf_a, buf_b)
        sems = (sem_a, sem_b)
        nchunk = CPW // half
        rd = pltpu.async_copy(x2d.at[pl.ds(base, half)], buf_a, sem_a)
        for t in range(nchunk):
            rd.wait()
            if t + 1 < nchunk:
                rd = pltpu.async_copy(
                    x2d.at[pl.ds(base + (t + 1) * half, half)],
                    bufs[(t + 1) % 2], sems[(t + 1) % 2])
            pltpu.sync_copy(bufs[t % 2], out.at[pl.ds(base + t * half, half)])
        plsc.subcore_barrier()
        pltpu.async_copy(rows.at[pl.ds(w * RPW, half)], buf_a, sem_a).wait()
        rd_b = pltpu.async_copy(rows.at[pl.ds(w * RPW + half, half)],
                                buf_b, sem_b)
        pltpu.async_copy(buf_a, out.at[idx_a], sem_a).wait()
        rd_b.wait()
        pltpu.async_copy(buf_b, out.at[idx_b], sem_b).wait()

    return gather_k, scatter_k


def _gather(x2d, idx2d):
    return _sc_kernels()[0](x2d, idx2d)


def _scatter(x2d, rows, idx2d):
    return _sc_kernels()[1](x2d, rows, idx2d)


def _mlp_body(fx_ref, w_ref, w1_ref, w3_ref, w2_ref, out_ref, fxb_ref):
    h = pl.program_id(0)

    @pl.when(h == 0)
    def _init():
        fxb_ref[...] = fx_ref[...].astype(jnp.bfloat16)
        out_ref[...] = jnp.zeros_like(out_ref)

    fxb = fxb_ref[...]
    h1 = jnp.dot(fxb, w1_ref[...].astype(jnp.bfloat16),
                 preferred_element_type=jnp.float32)
    h3 = jnp.dot(fxb, w3_ref[...].astype(jnp.bfloat16),
                 preferred_element_type=jnp.float32)
    g = (h1 * jax.nn.sigmoid(h1) * h3).astype(jnp.bfloat16)
    out_ref[...] += jnp.dot(g, w2_ref[...].astype(jnp.bfloat16),
                            preferred_element_type=jnp.float32)

    @pl.when(h == HBLK - 1)
    def _fin():
        out_ref[...] = fx_ref[...] + w_ref[...] * out_ref[...]


def _mlp(fx, wrow, w1, w3, w2):
    return pl.pallas_call(
        _mlp_body,
        grid=(HBLK,),
        in_specs=[
            pl.BlockSpec((NR, D), lambda h: (0, 0)),
            pl.BlockSpec((NR, 1), lambda h: (0, 0)),
            pl.BlockSpec((D, HB), lambda h: (0, h)),
            pl.BlockSpec((D, HB), lambda h: (0, h)),
            pl.BlockSpec((HB, D), lambda h: (h, 0)),
        ],
        out_specs=pl.BlockSpec((NR, D), lambda h: (0, 0)),
        out_shape=jax.ShapeDtypeStruct((NR, D), jnp.float32),
        scratch_shapes=[pltpu.VMEM((NR, D), jnp.bfloat16)],
        compiler_params=pltpu.CompilerParams(
            dimension_semantics=("arbitrary",)),
        interpret=_I,
    )(fx, wrow, w1, w3, w2)


def kernel(x, Wr, Wa, W1, W2, W3):
    wra = jnp.concatenate(
        [Wr, Wa, jnp.zeros((D, 126), jnp.float32)], axis=1).astype(jnp.bfloat16)
    sel8, w8, aux = _router_topk(x, wra)
    idx2d = sel8.reshape(NW, RPW)
    wrow = w8.reshape(NR, 1)
    x2d = x.reshape(B * S, D)
    fx = _gather(x2d, idx2d)
    rows = _mlp(fx, wrow, W1, W3, W2)
    out2d = _scatter(x2d, rows, idx2d)
    return out2d.reshape(B, S, D), aux.reshape(())


# speedup vs baseline: 8.7888x; 1.0007x over previous
import functools

import jax
import jax.numpy as jnp
from jax import lax
from jax.experimental import pallas as pl
from jax.experimental.pallas import tpu as pltpu
from jax.experimental.pallas import tpu_sc as plsc

B = 2
S = 4096
D = 2048
HID = 4 * D
K = 491
KP = 512
NR = B * KP
NC = 2
NS = 16
NW = NC * NS
RPW = NR // NW
CPW = (B * S) // NW
NBLK = 8
SB = S // NBLK
HBLK = 16
HB = HID // HBLK
NEG_INF = float("-inf")


def _router_topk_body(x_ref, wra_ref, sel_ref, w_ref, aux_ref,
                      lg_s, alog_s):
    i = pl.program_id(0)

    @pl.when(i < NBLK)
    def _step():
        xb = x_ref[...].reshape(B * SB, D).astype(jnp.bfloat16)
        y = jnp.dot(xb, wra_ref[...], preferred_element_type=jnp.float32)
        lg_s[:, pl.ds(i * SB, SB)] = y[:, 0].reshape(B, SB)
        alog_s[:, pl.ds(i * SB, SB)] = y[:, 1].reshape(B, SB)

    @pl.when(i == NBLK)
    def _fin():
        _topk_compute(lg_s, alog_s, sel_ref, w_ref, aux_ref)


def _router_topk(x, wra):
    return pl.pallas_call(
        _router_topk_body,
        grid=(NBLK + 1,),
        in_specs=[
            pl.BlockSpec((B, SB, D), lambda i: (0, jnp.minimum(i, NBLK - 1), 0)),
            pl.BlockSpec((D, 128), lambda i: (0, 0)),
        ],
        out_specs=[
            pl.BlockSpec((B, KP), lambda i: (0, 0)),
            pl.BlockSpec((B, KP), lambda i: (0, 0)),
            pl.BlockSpec((1, 1), lambda i: (0, 0)),
        ],
        out_shape=[
            jax.ShapeDtypeStruct((B, KP), jnp.int32),
            jax.ShapeDtypeStruct((B, KP), jnp.float32),
            jax.ShapeDtypeStruct((1, 1), jnp.float32),
        ],
        scratch_shapes=[
            pltpu.VMEM((B, S), jnp.float32),
            pltpu.VMEM((B, S), jnp.float32),
        ],
        compiler_params=pltpu.CompilerParams(
            dimension_semantics=("arbitrary",)),
    )(x, wra)


def _excl_prefix(m):
    tri128 = (lax.broadcasted_iota(jnp.int32, (128, 128), 0)
              < lax.broadcasted_iota(jnp.int32, (128, 128), 1)).astype(jnp.float32)
    in_row = jax.lax.dot(m, tri128, precision=lax.Precision.HIGHEST)
    row_tot = jnp.sum(m, axis=1, keepdims=True)
    tri32 = (lax.broadcasted_iota(jnp.int32, (32, 32), 0)
             < lax.broadcasted_iota(jnp.int32, (32, 32), 1)).astype(jnp.float32)
    row_off = jax.lax.dot(row_tot.reshape(1, 32), tri32,
                          precision=lax.Precision.HIGHEST)
    return in_row + row_off.reshape(32, 1)


def _topk_compute(lg_ref, alog_ref, sel_ref, w_ref, aux_ref):
    gif = (lax.broadcasted_iota(jnp.int32, (32, 128), 0) * 128
           + lax.broadcasted_iota(jnp.int32, (32, 128), 1)).astype(jnp.float32)
    piota = lax.broadcasted_iota(jnp.int32, (KP, 1), 0).astype(jnp.float32)
    piota_row = lax.broadcasted_iota(jnp.int32, (1, KP), 1).astype(jnp.float32)
    jcol = lax.broadcasted_iota(jnp.int32, (KP, 1), 0)
    pmask = (piota < K)
    pmask_row = (piota_row < K)

    masks = []
    for b in range(B):
        v = lg_ref[b, :].reshape(32, 128)
        s = lax.bitcast_convert_type(v, jnp.int32)
        key = s ^ ((s >> 31) & jnp.int32(0x7FFFFFFF))

        def bs_step(_, carry):
            lo, hi = carry
            mid = (lo >> 1) + (hi >> 1) + ((lo | hi) & 1)
            cnt = jnp.sum((key >= mid).astype(jnp.int32))
            big = cnt >= K
            return (jnp.where(big, mid, lo), jnp.where(big, hi, mid - 1))

        lo0 = jnp.int32(-2147483647 - 1)
        hi0 = jnp.int32(2147483647)
        thr, _ = lax.fori_loop(0, 32, bs_step, (lo0, hi0))

        gt = (key > thr).astype(jnp.float32)
        ties = (key == thr).astype(jnp.float32)
        m_rem = (jnp.float32(K) - jnp.sum(gt)).astype(jnp.float32)
        tie_pref = _excl_prefix(ties)
        m = gt + ties * (tie_pref < m_rem).astype(jnp.float32)
        masks.append(m)

        pos = _excl_prefix(m)
        u = jnp.min(jnp.where(m == 0, gif, jnp.float32(S)))
        posr = pos.reshape(1, S)
        mr = m.reshape(1, S)
        gir = gif.reshape(1, S)
        vr = v.reshape(1, S)
        sel_asc = jnp.zeros((KP, 1), jnp.float32)
        kv = jnp.zeros((KP, 1), jnp.float32)
        CS = 512
        for ci in range(S // CS):
            lo, hi = ci * CS, (ci + 1) * CS
            Ac = ((posr[:, lo:hi] == piota).astype(jnp.float32)
                  * mr[:, lo:hi])
            sel_asc = sel_asc + jnp.sum(Ac * gir[:, lo:hi], axis=1,
                                        keepdims=True)
            kv = kv + jnp.sum(Ac * vr[:, lo:hi], axis=1, keepdims=True)
        sel_p = jnp.where(pmask, sel_asc, u)
        vsel = jnp.where(pmask, kv, jnp.float32(NEG_INF))
        vrow = jnp.transpose(vsel)
        rank = jnp.zeros((KP, 1), jnp.float32)
        for qc in range(KP // 128):
            vq = vrow[:, qc * 128:(qc + 1) * 128]
            qid = (lax.broadcasted_iota(jnp.int32, (1, 128), 1)
                   + qc * 128)
            cmp = ((vq > vsel) | ((vq == vsel) & (qid < jcol)))
            rank = rank + jnp.sum(cmp.astype(jnp.float32), axis=1,
                                  keepdims=True)
        wrow = jnp.zeros((1, KP), jnp.float32)
        for jc in range(KP // 128):
            rj = rank[jc * 128:(jc + 1) * 128, :]
            vj = vsel[jc * 128:(jc + 1) * 128, :]
            oh = (rj == piota_row).astype(jnp.float32)
            wrow = wrow + jnp.sum(oh * vj, axis=0, keepdims=True)
        wv = jnp.where(pmask_row, wrow, 0.0)
        sel_row = jnp.transpose(sel_p).astype(jnp.int32) + b * S
        sel_ref[pl.ds(b, 1), :] = sel_row
        w_ref[pl.ds(b, 1), :] = wv

    al = alog_ref[...]
    p = jnp.clip(jax.nn.sigmoid(al), 1e-7, 1.0 - 1e-7)
    logp = jnp.log(p)
    log1mp = jnp.log(1.0 - p)
    base = jnp.sum(log1mp)
    mu = jnp.minimum(masks[0] + masks[1], 1.0)
    c0 = (logp[0, :] - log1mp[0, :]).reshape(32, 128)
    corr = jnp.sum(mu * c0)
    aux_ref[...] = (-(base + corr) / jnp.float32(B * S)).reshape(1, 1)


@functools.lru_cache(maxsize=None)
def _sc_kernels():
    mesh = plsc.VectorSubcoreMesh(core_axis_name="c", subcore_axis_name="s",
                                  num_cores=NC, num_subcores=NS)

    @functools.partial(
        pl.kernel,
        out_type=jax.ShapeDtypeStruct((NR, D), jnp.float32),
        mesh=mesh,
        scratch_types=[
            pltpu.VMEM((RPW,), jnp.int32),
            pltpu.VMEM((RPW, D), jnp.float32),
            pltpu.SemaphoreType.DMA,
        ],
    )
    def gather_k(x2d, idx2d, fx, idx_v, rows_v, sem):
        w = lax.axis_index("c") * NS + lax.axis_index("s")
        pltpu.sync_copy(idx2d.at[w], idx_v)
        pltpu.async_copy(x2d.at[idx_v], rows_v, sem).wait()
        pltpu.sync_copy(rows_v, fx.at[pl.ds(w * RPW, RPW)])

    @functools.partial(
        pl.kernel,
        out_type=jax.ShapeDtypeStruct((B * S, D), jnp.float32),
        mesh=mesh,
        scratch_types=[
            pltpu.VMEM((RPW // 2,), jnp.int32),
            pltpu.VMEM((RPW // 2,), jnp.int32),
            pltpu.VMEM((RPW // 2, D), jnp.float32),
            pltpu.VMEM((RPW // 2, D), jnp.float32),
            pltpu.SemaphoreType.DMA,
            pltpu.SemaphoreType.DMA,
        ],
    )
    def scatter_k(x2d, rows, idx2d, out, idx_a, idx_b, buf_a, buf_b,
                  sem_a, sem_b):
        c = lax.axis_index("c")
        w = c * NS + lax.axis_index("s")
        base = w * CPW
        half = RPW // 2
        pltpu.sync_copy(idx2d.at[w, pl.ds(0, half)], idx_a)
        pltpu.sync_copy(idx2d.at[w, pl.ds(half, half)], idx_b)
        bufs = (buf_a, buf_b)
        sems = (sem_a, sem_b)
        nchunk = CPW // half
        rd = pltpu.async_copy(x2d.at[pl.ds(base, half)], buf_a, sem_a)
        for t in range(nchunk):
            rd.wait()
            if t + 1 < nchunk:
                rd = pltpu.async_copy(
                    x2d.at[pl.ds(base + (t + 1) * half, half)],
                    bufs[(t + 1) % 2], sems[(t + 1) % 2])
            pltpu.sync_copy(bufs[t % 2], out.at[pl.ds(base + t * half, half)])
        plsc.subcore_barrier()
        pltpu.async_copy(rows.at[pl.ds(w * RPW, half)], buf_a, sem_a).wait()
        rd_b = pltpu.async_copy(rows.at[pl.ds(w * RPW + half, half)],
                                buf_b, sem_b)
        pltpu.async_copy(buf_a, out.at[idx_a], sem_a).wait()
        rd_b.wait()
        pltpu.async_copy(buf_b, out.at[idx_b], sem_b).wait()

    return gather_k, scatter_k


def _gather(x2d, idx2d):
    return _sc_kernels()[0](x2d, idx2d)


def _scatter(x2d, rows, idx2d):
    return _sc_kernels()[1](x2d, rows, idx2d)


def _mlp_body(fx_ref, w_ref, w1_ref, w3_ref, w2_ref, out_ref, fxb_ref):
    h = pl.program_id(0)

    @pl.when(h == 0)
    def _init():
        fxb_ref[...] = fx_ref[...].astype(jnp.bfloat16)
        out_ref[...] = jnp.zeros_like(out_ref)

    fxb = fxb_ref[...]
    h1 = jnp.dot(fxb, w1_ref[...].astype(jnp.bfloat16),
                 preferred_element_type=jnp.float32)
    h3 = jnp.dot(fxb, w3_ref[...].astype(jnp.bfloat16),
                 preferred_element_type=jnp.float32)
    g = (h1 * jax.nn.sigmoid(h1) * h3).astype(jnp.bfloat16)
    out_ref[...] += jnp.dot(g, w2_ref[...].astype(jnp.bfloat16),
                            preferred_element_type=jnp.float32)

    @pl.when(h == HBLK - 1)
    def _fin():
        out_ref[...] = fx_ref[...] + w_ref[...] * out_ref[...]


def _mlp(fx, wrow, w1, w3, w2):
    return pl.pallas_call(
        _mlp_body,
        grid=(HBLK,),
        in_specs=[
            pl.BlockSpec((NR, D), lambda h: (0, 0)),
            pl.BlockSpec((NR, 1), lambda h: (0, 0)),
            pl.BlockSpec((D, HB), lambda h: (0, h)),
            pl.BlockSpec((D, HB), lambda h: (0, h)),
            pl.BlockSpec((HB, D), lambda h: (h, 0)),
        ],
        out_specs=pl.BlockSpec((NR, D), lambda h: (0, 0)),
        out_shape=jax.ShapeDtypeStruct((NR, D), jnp.float32),
        scratch_shapes=[pltpu.VMEM((NR, D), jnp.bfloat16)],
        compiler_params=pltpu.CompilerParams(
            dimension_semantics=("arbitrary",)),
    )(fx, wrow, w1, w3, w2)


def kernel(x, Wr, Wa, W1, W2, W3):
    wra = jnp.concatenate(
        [Wr, Wa, jnp.zeros((D, 126), jnp.float32)], axis=1).astype(jnp.bfloat16)
    sel8, w8, aux = _router_topk(x, wra)
    idx2d = sel8.reshape(NW, RPW)
    wrow = w8.reshape(NR, 1)
    x2d = x.reshape(B * S, D)
    fx = _gather(x2d, idx2d)
    rows = _mlp(fx, wrow, W1, W3, W2)
    out2d = _scatter(x2d, rows, idx2d)
    return out2d.reshape(B, S, D), aux.reshape(())
